# baseline (speedup 1.0000x reference)
import dataclasses

import jax
import jax.numpy as jnp
from jax import lax
from jax.experimental import pallas as pl
from jax.experimental.pallas import tpu as pltpu
from jax.experimental.pallas import tpu_sc as plsc

N_NODES = 10000
N_EDGES = 320000
N_HEADS = 8
HEAD_DIM = 16
FEAT = N_HEADS * HEAD_DIM

NW = 32
NPW = 320
N_PAD = NW * NPW
RPW = NPW + 16
RP_PAD = (NW - 1) * NPW + RPW
EW = 256
TRASH = NPW

_GATHER_DNUMS = lax.GatherDimensionNumbers(
    offset_dims=(), collapsed_slice_dims=(0,), start_index_map=(0,))


def _splat(vec, lane):
  idx = jnp.full((16,), lane, jnp.int32)
  return lax.gather(vec, idx[:, None], _GATHER_DNUMS, slice_sizes=(1,),
                    mode=lax.GatherScatterMode.PROMISE_IN_BOUNDS)


def _scalar_at(ref, off):
  chunk = ref[pl.ds(off, 16)]
  lane = lax.broadcasted_iota(jnp.int32, (16,), 0)
  return jnp.sum(jnp.where(lane == 0, chunk, 0))


def _sc_kernel_body(rp_hbm, col_hbm, w_hbm, nf_hbm, out_hbm,
                    rp_v, idx0, idx1, rows0, rows1, w0, w1, seg_v, dst_v,
                    acc_v, scol0, scol1, sg0a, sg0b, sg1a, sg1b, sw0, sw1):
  idx = [idx0, idx1]
  rows = [rows0, rows1]
  wbf = [w0, w1]
  scol = [scol0, scol1]
  sga = [sg0a, sg1a]
  sgb = [sg0b, sg1b]
  sw = [sw0, sw1]

  lane = lax.broadcasted_iota(jnp.int32, (16,), 0)
  zeros_f = jnp.zeros((16,), jnp.float32)
  zeros_i = jnp.zeros((16,), jnp.int32)
  ones_i = jnp.full((16,), 1, jnp.int32)

  wid = lax.axis_index("c") * 16 + lax.axis_index("s")
  n0 = pl.multiple_of(wid * NPW, 64)

  rp_cp = pltpu.async_copy(rp_hbm.at[pl.ds(n0, RPW)], rp_v, sw0)

  @plsc.parallel_loop(0, NPW + 1)
  def _zero(r):
    for h in range(N_HEADS):
      acc_v[r, pl.ds(16 * h, 16)] = zeros_f

  rp_cp.wait()

  e0 = _scalar_at(rp_v, 0)
  e1 = _scalar_at(rp_v, NPW)
  eb0 = e0 - lax.rem(e0, EW)
  nwin = lax.shift_right_logical(e1 - eb0 + (EW - 1), 8)

  def wbase_of(k):
    return pl.multiple_of(eb0 + EW * k, EW)

  def dma_base_of(k):
    return pl.multiple_of(jnp.minimum(wbase_of(k), N_EDGES - EW), EW)

  def start_col(k, b):
    pltpu.async_copy(col_hbm.at[pl.ds(dma_base_of(k), EW)], idx[b], scol[b])

  def wait_col(b):
    pltpu.make_async_copy(col_hbm.at[pl.ds(0, EW)], idx[b], scol[b]).wait()

  def start_main(k, b):
    pltpu.async_copy(nf_hbm.at[idx[b]], rows[b], sga[b])
    pltpu.async_copy(
        w_hbm.at[pl.ds(pl.multiple_of(N_HEADS * dma_base_of(k), N_HEADS * EW),
                       N_HEADS * EW)], wbf[b].at[pl.ds(0, N_HEADS * EW)],
        sw[b])

  def wait_main(b):
    pltpu.make_async_copy(nf_hbm.at[idx[b]], rows[b], sga[b]).wait()
    pltpu.make_async_copy(w_hbm.at[pl.ds(0, N_HEADS * EW)],
                          wbf[b].at[pl.ds(0, N_HEADS * EW)], sw[b]).wait()

  def seg_compute(wbase, carry):
    for c in range(EW // 16):
      seg_v[pl.ds(16 * c, 16)] = zeros_i
    for c in range(RPW // 16):
      rpc = rp_v[pl.ds(16 * c, 16)]
      pos = rpc - wbase
      nid = 16 * c + lane
      m = (pos >= 0) & (pos < EW) & (nid <= NPW)
      plsc.addupdate_scatter(seg_v, [pos], ones_i, mask=m)

    def seg_body(c, cin):
      s = seg_v[pl.ds(16 * c, 16)]
      cs = plsc.cumsum(s) + cin
      d = cs - 1
      d = jnp.where((d < 0) | (d >= NPW), TRASH, d)
      dst_v[pl.ds(16 * c, 16)] = d
      return cin + jnp.sum(s)

    return lax.fori_loop(0, EW // 16, seg_body, carry)

  def edge_loop(rows_ref, w_ref):
    @plsc.parallel_loop(0, EW, unroll=2)
    def _edges(e):
      dsplat = plsc.load_gather(dst_v, [jnp.broadcast_to(e, (16,))])
      wv = w_ref[pl.ds(8 * e, 16)]
      for h in range(N_HEADS):
        wsplat = _splat(wv, h)
        chunk = rows_ref[e, pl.ds(16 * h, 16)]
        plsc.addupdate_scatter(acc_v, [dsplat, lane + 16 * h],
                               chunk * wsplat)

  def step(k, carry, cur, nxt):
    wait_col(nxt)
    start_main(k + 1, nxt)
    carry = seg_compute(wbase_of(k), carry)
    wait_main(cur)
    start_col(k + 2, cur)
    edge_loop(rows[cur], wbf[cur])
    return carry

  start_col(0, 0)
  wait_col(0)
  start_main(0, 0)
  start_col(1, 1)

  def pair_body(i, carry):
    carry = step(2 * i, carry, 0, 1)
    carry = step(2 * i + 1, carry, 1, 0)
    return carry

  lax.fori_loop(0, lax.shift_right_logical(nwin + 1, 1), pair_body,
                jnp.int32(0))

  wait_main(0)
  wait_col(1)

  last = N_NODES - (NW - 1) * NPW

  @pl.when(wid < NW - 1)
  def _full():
    pltpu.async_copy(acc_v.at[pl.ds(0, NPW)], out_hbm.at[pl.ds(n0, NPW)],
                     scol0).wait()

  @pl.when(wid == NW - 1)
  def _tail():
    pltpu.async_copy(acc_v.at[pl.ds(0, last)],
                     out_hbm.at[pl.ds((NW - 1) * NPW, last)], scol0).wait()


def kernel(row_ptr, col_idx, edge_weight, node_feat):
  rp = row_ptr.astype(jnp.int32)
  rp_pad = jnp.concatenate(
      [rp, jnp.full((RP_PAD - (N_NODES + 1),), rp[-1], jnp.int32)])
  col_flat = col_idx.astype(jnp.int32)
  w_flat = edge_weight.reshape(-1)
  nf2d = node_feat.reshape(N_NODES, FEAT)

  mesh = plsc.VectorSubcoreMesh(core_axis_name="c", subcore_axis_name="s")
  cp = pltpu.CompilerParams()
  if "needs_layout_passes" in pltpu.CompilerParams.__dataclass_fields__:
    cp = dataclasses.replace(cp, needs_layout_passes=False)
  kfn = pl.kernel(
      _sc_kernel_body,
      out_type=jax.ShapeDtypeStruct((N_NODES, FEAT), jnp.float32),
      mesh=mesh,
      compiler_params=cp,
      scratch_types=[
          pltpu.VMEM((RPW,), jnp.int32),
          pltpu.VMEM((EW,), jnp.int32),
          pltpu.VMEM((EW,), jnp.int32),
          pltpu.VMEM((EW, FEAT), jnp.float32),
          pltpu.VMEM((EW, FEAT), jnp.float32),
          pltpu.VMEM((N_HEADS * EW + 16,), jnp.float32),
          pltpu.VMEM((N_HEADS * EW + 16,), jnp.float32),
          pltpu.VMEM((EW,), jnp.int32),
          pltpu.VMEM((EW,), jnp.int32),
          pltpu.VMEM((NPW + 1, FEAT), jnp.float32),
          pltpu.SemaphoreType.DMA,
          pltpu.SemaphoreType.DMA,
          pltpu.SemaphoreType.DMA,
          pltpu.SemaphoreType.DMA,
          pltpu.SemaphoreType.DMA,
          pltpu.SemaphoreType.DMA,
          pltpu.SemaphoreType.DMA,
          pltpu.SemaphoreType.DMA,
      ],
  )
  out = kfn(rp_pad, col_flat, w_flat, nf2d)
  return out.reshape(N_NODES, N_HEADS, HEAD_DIM)


# speedup vs baseline: 1.0022x; 1.0022x over previous
import dataclasses

import jax
import jax.numpy as jnp
from jax import lax
from jax.experimental import pallas as pl
from jax.experimental.pallas import tpu as pltpu
from jax.experimental.pallas import tpu_sc as plsc

N_NODES = 10000
N_EDGES = 320000
N_HEADS = 8
HEAD_DIM = 16
FEAT = N_HEADS * HEAD_DIM

NW = 32
NPW = 320
N_PAD = NW * NPW
RPW = NPW + 16
RP_PAD = (NW - 1) * NPW + RPW
EW = 256
TRASH = NPW

_GATHER_DNUMS = lax.GatherDimensionNumbers(
    offset_dims=(), collapsed_slice_dims=(0,), start_index_map=(0,))


def _splat(vec, lane):
  idx = jnp.full((16,), lane, jnp.int32)
  return lax.gather(vec, idx[:, None], _GATHER_DNUMS, slice_sizes=(1,),
                    mode=lax.GatherScatterMode.PROMISE_IN_BOUNDS)


def _scalar_at(ref, off):
  chunk = ref[pl.ds(off, 16)]
  lane = lax.broadcasted_iota(jnp.int32, (16,), 0)
  return jnp.sum(jnp.where(lane == 0, chunk, 0))


def _sc_kernel_body(rp_hbm, col_hbm, w_hbm, nf_hbm, out_hbm,
                    rp_v, idx0, idx1, rows0, rows1, w0, w1, seg_v, dst_v,
                    acc_v, scol0, scol1, sg0a, sg0b, sg1a, sg1b, sw0, sw1):
  idx = [idx0, idx1]
  rows = [rows0, rows1]
  wbf = [w0, w1]
  scol = [scol0, scol1]
  sga = [sg0a, sg1a]
  sgb = [sg0b, sg1b]
  sw = [sw0, sw1]

  lane = lax.broadcasted_iota(jnp.int32, (16,), 0)
  zeros_f = jnp.zeros((16,), jnp.float32)
  zeros_i = jnp.zeros((16,), jnp.int32)
  ones_i = jnp.full((16,), 1, jnp.int32)

  wid = lax.axis_index("c") * 16 + lax.axis_index("s")
  n0 = pl.multiple_of(wid * NPW, 64)

  rp_cp = pltpu.async_copy(rp_hbm.at[pl.ds(n0, RPW)], rp_v, sw0)

  @plsc.parallel_loop(0, NPW + 1)
  def _zero(r):
    for h in range(N_HEADS):
      acc_v[r, pl.ds(16 * h, 16)] = zeros_f

  rp_cp.wait()

  e0 = _scalar_at(rp_v, 0)
  e1 = _scalar_at(rp_v, NPW)
  eb0 = e0 - lax.rem(e0, EW)
  nwin = lax.shift_right_logical(e1 - eb0 + (EW - 1), 8)

  def wbase_of(k):
    return pl.multiple_of(eb0 + EW * k, EW)

  def dma_base_of(k):
    return pl.multiple_of(jnp.minimum(wbase_of(k), N_EDGES - EW), EW)

  def start_col(k, b):
    pltpu.async_copy(col_hbm.at[pl.ds(dma_base_of(k), EW)], idx[b], scol[b])

  def wait_col(b):
    pltpu.make_async_copy(col_hbm.at[pl.ds(0, EW)], idx[b], scol[b]).wait()

  def start_main(k, b):
    pltpu.async_copy(nf_hbm.at[idx[b]], rows[b], sga[b])
    pltpu.async_copy(
        w_hbm.at[pl.ds(pl.multiple_of(N_HEADS * dma_base_of(k), N_HEADS * EW),
                       N_HEADS * EW)], wbf[b].at[pl.ds(0, N_HEADS * EW)],
        sw[b])

  def wait_main(b):
    pltpu.make_async_copy(nf_hbm.at[idx[b]], rows[b], sga[b]).wait()
    pltpu.make_async_copy(w_hbm.at[pl.ds(0, N_HEADS * EW)],
                          wbf[b].at[pl.ds(0, N_HEADS * EW)], sw[b]).wait()

  def seg_compute(wbase, carry):
    for c in range(EW // 16):
      seg_v[pl.ds(16 * c, 16)] = zeros_i
    for c in range(RPW // 16):
      rpc = rp_v[pl.ds(16 * c, 16)]
      pos = rpc - wbase
      nid = 16 * c + lane
      m = (pos >= 0) & (pos < EW) & (nid <= NPW)
      plsc.addupdate_scatter(seg_v, [pos], ones_i, mask=m)

    def seg_body(c, cin):
      s = seg_v[pl.ds(16 * c, 16)]
      cs = plsc.cumsum(s) + cin
      d = cs - 1
      d = jnp.where((d < 0) | (d >= NPW), TRASH, d)
      dst_v[pl.ds(16 * c, 16)] = d
      return cin + jnp.sum(s)

    return lax.fori_loop(0, EW // 16, seg_body, carry)

  def edge_loop(rows_ref, w_ref):
    @plsc.parallel_loop(0, EW, unroll=4)
    def _edges(e):
      dsplat = plsc.load_gather(dst_v, [jnp.broadcast_to(e, (16,))])
      wv = w_ref[pl.ds(8 * e, 16)]
      for h in range(N_HEADS):
        wsplat = _splat(wv, h)
        chunk = rows_ref[e, pl.ds(16 * h, 16)]
        plsc.addupdate_scatter(acc_v, [dsplat, lane + 16 * h],
                               chunk * wsplat)

  def step(k, carry, cur, nxt):
    wait_col(nxt)
    start_main(k + 1, nxt)
    carry = seg_compute(wbase_of(k), carry)
    wait_main(cur)
    start_col(k + 2, cur)
    edge_loop(rows[cur], wbf[cur])
    return carry

  start_col(0, 0)
  wait_col(0)
  start_main(0, 0)
  start_col(1, 1)

  def pair_body(i, carry):
    carry = step(2 * i, carry, 0, 1)
    carry = step(2 * i + 1, carry, 1, 0)
    return carry

  lax.fori_loop(0, lax.shift_right_logical(nwin + 1, 1), pair_body,
                jnp.int32(0))

  wait_main(0)
  wait_col(1)

  last = N_NODES - (NW - 1) * NPW

  @pl.when(wid < NW - 1)
  def _full():
    pltpu.async_copy(acc_v.at[pl.ds(0, NPW)], out_hbm.at[pl.ds(n0, NPW)],
                     scol0).wait()

  @pl.when(wid == NW - 1)
  def _tail():
    pltpu.async_copy(acc_v.at[pl.ds(0, last)],
                     out_hbm.at[pl.ds((NW - 1) * NPW, last)], scol0).wait()


def kernel(row_ptr, col_idx, edge_weight, node_feat):
  rp = row_ptr.astype(jnp.int32)
  rp_pad = jnp.concatenate(
      [rp, jnp.full((RP_PAD - (N_NODES + 1),), rp[-1], jnp.int32)])
  col_flat = col_idx.astype(jnp.int32)
  w_flat = edge_weight.reshape(-1)
  nf2d = node_feat.reshape(N_NODES, FEAT)

  mesh = plsc.VectorSubcoreMesh(core_axis_name="c", subcore_axis_name="s")
  cp = pltpu.CompilerParams()
  if "needs_layout_passes" in pltpu.CompilerParams.__dataclass_fields__:
    cp = dataclasses.replace(cp, needs_layout_passes=False)
  kfn = pl.kernel(
      _sc_kernel_body,
      out_type=jax.ShapeDtypeStruct((N_NODES, FEAT), jnp.float32),
      mesh=mesh,
      compiler_params=cp,
      scratch_types=[
          pltpu.VMEM((RPW,), jnp.int32),
          pltpu.VMEM((EW,), jnp.int32),
          pltpu.VMEM((EW,), jnp.int32),
          pltpu.VMEM((EW, FEAT), jnp.float32),
          pltpu.VMEM((EW, FEAT), jnp.float32),
          pltpu.VMEM((N_HEADS * EW + 16,), jnp.float32),
          pltpu.VMEM((N_HEADS * EW + 16,), jnp.float32),
          pltpu.VMEM((EW,), jnp.int32),
          pltpu.VMEM((EW,), jnp.int32),
          pltpu.VMEM((NPW + 1, FEAT), jnp.float32),
          pltpu.SemaphoreType.DMA,
          pltpu.SemaphoreType.DMA,
          pltpu.SemaphoreType.DMA,
          pltpu.SemaphoreType.DMA,
          pltpu.SemaphoreType.DMA,
          pltpu.SemaphoreType.DMA,
          pltpu.SemaphoreType.DMA,
          pltpu.SemaphoreType.DMA,
      ],
  )
  out = kfn(rp_pad, col_flat, w_flat, nf2d)
  return out.reshape(N_NODES, N_HEADS, HEAD_DIM)
